# baseline (speedup 1.0000x reference)
import functools

import jax
import jax.numpy as jnp
from jax import lax
from jax.experimental import pallas as pl
from jax.experimental.pallas import tpu as pltpu
from jax.experimental.pallas import tpu_sc as plsc

N_NODES = 10000
N_EDGES = 320000
D_EDGE = 16
EMB = 128

NC = 2
NS = 16
NW = NC * NS
CH = 128
NCHUNK = N_EDGES // CH
NPART = 2
PCHUNK = NCHUNK // NPART
E_P = PCHUNK * CH
KFULL = PCHUNK // NW
NEXTRA = PCHUNK - KFULL * NW
NB = 2
NFULL = KFULL // NB
NP = 10240
RPT = NP // NS

BE = 6400
BN = 2000


def _edge_body(attr_ref, msg_ref, w_ref, b_ref, o_ref):
    emb = lax.dot_general(attr_ref[...], w_ref[...],
                          dimension_numbers=(((0,), (0,)), ((), ())),
                          preferred_element_type=jnp.float32)
    o_ref[...] = (emb + b_ref[...]) * msg_ref[...]


def _edge_stage(edge_attr_t, msg_emb, W_edge, b_edge, part):
    offb = part * (E_P // BE)
    return pl.pallas_call(
        _edge_body,
        grid=(E_P // BE,),
        in_specs=[
            pl.BlockSpec((D_EDGE, BE), lambda i, offb=offb: (0, i + offb)),
            pl.BlockSpec((BE, EMB), lambda i, offb=offb: (i + offb, 0)),
            pl.BlockSpec((D_EDGE, EMB), lambda i: (0, 0)),
            pl.BlockSpec((1, EMB), lambda i: (0, 0)),
        ],
        out_specs=pl.BlockSpec((BE, EMB), lambda i: (i, 0)),
        out_shape=jax.ShapeDtypeStruct((E_P, EMB), jnp.float32),
    )(edge_attr_t, msg_emb, W_edge, b_edge)


def _make_sc_body(part):
    lo = part * PCHUNK

    def _sc_body(x_hbm, eidx_hbm, zeros_hbm, out_hbm, xbs, idbs, acc, xsems, isems):
        c = lax.axis_index("c")
        s = lax.axis_index("s")
        wid = s * NC + c

        def chunk_of(k):
            return lax.select(k < KFULL, k * NW + wid, KFULL * NW + wid)

        def start_fetch(k, t):
            ch = chunk_of(k)
            pltpu.async_copy(x_hbm.at[pl.ds(ch * CH, CH)], xbs[t], xsems[t])
            pltpu.async_copy(
                eidx_hbm.at[:, pl.ds((lo + ch) * CH, CH)], idbs[t], isems[t])

        def finish_chunk(k, t):
            ch = chunk_of(k)
            pltpu.make_async_copy(
                x_hbm.at[pl.ds(ch * CH, CH)], xbs[t], xsems[t]).wait()
            pltpu.make_async_copy(
                eidx_hbm.at[:, pl.ds((lo + ch) * CH, CH)], idbs[t], isems[t]).wait()
            pltpu.sync_copy(xbs[t], acc.at[idbs[t].at[1]], add=True)

        nk = KFULL + lax.select(wid < NEXTRA, 1, 0)

        for t in range(NB):
            start_fetch(t, t)
        pltpu.sync_copy(zeros_hbm, acc.at[pl.ds(s * RPT, RPT)])
        plsc.subcore_barrier()

        def body(i, carry):
            for t in range(NB):
                k = i * NB + t
                finish_chunk(k, t)

                @pl.when(k + NB < nk)
                def _():
                    start_fetch(k + NB, t)

            return carry

        lax.fori_loop(0, NFULL, body, 0)
        for k in range(NFULL * NB, KFULL):
            finish_chunk(k, k % NB)

        @pl.when(nk > KFULL)
        def _():
            finish_chunk(KFULL, KFULL % NB)

        plsc.subcore_barrier()
        pltpu.sync_copy(acc.at[pl.ds(s * RPT, RPT)],
                        out_hbm.at[c, pl.ds(s * RPT, RPT)])

    return _sc_body


@functools.cache
def _sc_scatter(part):
    return pl.kernel(
        _make_sc_body(part),
        out_type=jax.ShapeDtypeStruct((NC, NP, EMB), jnp.float32),
        mesh=plsc.VectorSubcoreMesh(
            core_axis_name="c", subcore_axis_name="s", num_cores=NC, num_subcores=NS
        ),
        scratch_types=[
            tuple(pltpu.VMEM((CH, EMB), jnp.float32) for _ in range(NB)),
            tuple(pltpu.VMEM((2, CH), jnp.int32) for _ in range(NB)),
            pltpu.VMEM_SHARED((NP, EMB), jnp.float32),
            tuple(pltpu.SemaphoreType.DMA for _ in range(NB)),
            tuple(pltpu.SemaphoreType.DMA for _ in range(NB)),
        ],
    )


def _mlp_body(pa0_ref, pa1_ref, pb0_ref, pb1_ref, w0_ref, b0_ref, w4_ref, o_ref):
    h = pa0_ref[0] + pa1_ref[0] + pb0_ref[0] + pb1_ref[0]
    w0 = w0_ref[...]
    b0 = b0_ref[...]
    x1 = jnp.maximum(jnp.dot(h, w0, preferred_element_type=jnp.float32) + b0, 0.0)
    x2 = jnp.maximum(jnp.dot(x1, w0, preferred_element_type=jnp.float32) + b0, 0.0)
    x3 = jnp.maximum(jnp.dot(x2, w0, preferred_element_type=jnp.float32) + b0, 0.0)
    o_ref[...] = jnp.dot(x3, w4_ref[...], preferred_element_type=jnp.float32)


def _mlp_stage(pa, pb, W0, b0, W4):
    return pl.pallas_call(
        _mlp_body,
        grid=(N_NODES // BN,),
        in_specs=[
            pl.BlockSpec((1, BN, EMB), lambda i: (0, i, 0)),
            pl.BlockSpec((1, BN, EMB), lambda i: (1, i, 0)),
            pl.BlockSpec((1, BN, EMB), lambda i: (0, i, 0)),
            pl.BlockSpec((1, BN, EMB), lambda i: (1, i, 0)),
            pl.BlockSpec((EMB, EMB), lambda i: (0, 0)),
            pl.BlockSpec((1, EMB), lambda i: (0, 0)),
            pl.BlockSpec((EMB, EMB), lambda i: (0, 0)),
        ],
        out_specs=pl.BlockSpec((BN, EMB), lambda i: (i, 0)),
        out_shape=jax.ShapeDtypeStruct((N_NODES, EMB), jnp.float32),
    )(pa, pa, pb, pb, W0, b0, W4)


def kernel(edge_attr, edge_index, msg_emb, num_nodes, W_edge, b_edge, W0, b0, W4):
    attr_t = edge_attr.T
    b_e = b_edge.reshape(1, EMB)
    zeros = jnp.zeros((RPT, EMB), dtype=jnp.float32)
    xa = _edge_stage(attr_t, msg_emb, W_edge, b_e, 0)
    pa = _sc_scatter(0)(xa, edge_index, zeros)
    xb = _edge_stage(attr_t, msg_emb, W_edge, b_e, 1)
    pb = _sc_scatter(1)(xb, edge_index, zeros)
    return _mlp_stage(pa, pb, W0, b0.reshape(1, EMB), W4)


# speedup vs baseline: 1.0243x; 1.0243x over previous
import functools

import jax
import jax.numpy as jnp
from jax import lax
from jax.experimental import pallas as pl
from jax.experimental.pallas import tpu as pltpu
from jax.experimental.pallas import tpu_sc as plsc

N_NODES = 10000
N_EDGES = 320000
D_EDGE = 16
EMB = 128

NC = 2
NS = 16
NW = NC * NS
CH = 128
NCHUNK = N_EDGES // CH
KFULL = NCHUNK // NW
NEXTRA = NCHUNK - KFULL * NW
NB = 2
NFULL = KFULL // NB
NP = 10240
RPT = NP // NS
NLANE = 16

BE = 6400
BN = 2000


def _edge_body(attr_ref, msg_ref, w_ref, b_ref, o_ref):
    emb = lax.dot_general(attr_ref[...], w_ref[...],
                          dimension_numbers=(((0,), (0,)), ((), ())),
                          preferred_element_type=jnp.float32)
    o_ref[...] = (emb + b_ref[...]) * msg_ref[...]


def _edge_stage(edge_attr_t, msg_emb, W_edge, b_edge):
    return pl.pallas_call(
        _edge_body,
        grid=(N_EDGES // BE,),
        in_specs=[
            pl.BlockSpec((D_EDGE, BE), lambda i: (0, i)),
            pl.BlockSpec((BE, EMB), lambda i: (i, 0)),
            pl.BlockSpec((D_EDGE, EMB), lambda i: (0, 0)),
            pl.BlockSpec((1, EMB), lambda i: (0, 0)),
        ],
        out_specs=pl.BlockSpec((BE, EMB), lambda i: (i, 0)),
        out_shape=jax.ShapeDtypeStruct((N_EDGES, EMB), jnp.float32),
    )(edge_attr_t, msg_emb, W_edge, b_edge)


def _sc_body(x_hbm, eidx_hbm, out_hbm, xbs, idbs, acc, xsems, isems):
    c = lax.axis_index("c")
    s = lax.axis_index("s")
    wid = s * NC + c

    def chunk_of(k):
        return lax.select(k < KFULL, k * NW + wid, KFULL * NW + wid)

    def start_fetch(k, t):
        ch = chunk_of(k)
        pltpu.async_copy(x_hbm.at[pl.ds(ch * CH, CH)], xbs[t], xsems[t])
        pltpu.async_copy(eidx_hbm.at[:, pl.ds(ch * CH, CH)], idbs[t], isems[t])

    def finish_chunk(k, t):
        ch = chunk_of(k)
        pltpu.make_async_copy(
            x_hbm.at[pl.ds(ch * CH, CH)], xbs[t], xsems[t]).wait()
        pltpu.make_async_copy(
            eidx_hbm.at[:, pl.ds(ch * CH, CH)], idbs[t], isems[t]).wait()
        pltpu.sync_copy(xbs[t], acc.at[idbs[t].at[1]], add=True)

    nk = KFULL + lax.select(wid < NEXTRA, 1, 0)

    zv = jnp.zeros((NLANE,), jnp.float32)

    def zrow(r, carry):
        for j in range(EMB // NLANE):
            xbs[0][r, pl.ds(j * NLANE, NLANE)] = zv
        return carry

    lax.fori_loop(0, CH, zrow, 0)
    for rep in range(RPT // CH):
        pltpu.sync_copy(xbs[0], acc.at[pl.ds(s * RPT + rep * CH, CH)])

    for t in range(NB):
        start_fetch(t, t)
    plsc.subcore_barrier()

    def body(i, carry):
        for t in range(NB):
            k = i * NB + t
            finish_chunk(k, t)

            @pl.when(k + NB < nk)
            def _():
                start_fetch(k + NB, t)

        return carry

    lax.fori_loop(0, NFULL, body, 0)

    @pl.when(nk > KFULL)
    def _():
        finish_chunk(KFULL, KFULL % NB)

    plsc.subcore_barrier()
    pltpu.sync_copy(acc.at[pl.ds(s * RPT, RPT)],
                    out_hbm.at[c, pl.ds(s * RPT, RPT)])


@functools.cache
def _sc_scatter():
    return pl.kernel(
        _sc_body,
        out_type=jax.ShapeDtypeStruct((NC, NP, EMB), jnp.float32),
        mesh=plsc.VectorSubcoreMesh(
            core_axis_name="c", subcore_axis_name="s", num_cores=NC, num_subcores=NS
        ),
        scratch_types=[
            tuple(pltpu.VMEM((CH, EMB), jnp.float32) for _ in range(NB)),
            tuple(pltpu.VMEM((2, CH), jnp.int32) for _ in range(NB)),
            pltpu.VMEM_SHARED((NP, EMB), jnp.float32),
            tuple(pltpu.SemaphoreType.DMA for _ in range(NB)),
            tuple(pltpu.SemaphoreType.DMA for _ in range(NB)),
        ],
    )


def _mlp_body(p0_ref, p1_ref, w0_ref, b0_ref, w4_ref, o_ref):
    h = p0_ref[0] + p1_ref[0]
    w0 = w0_ref[...]
    b0 = b0_ref[...]
    x1 = jnp.maximum(jnp.dot(h, w0, preferred_element_type=jnp.float32) + b0, 0.0)
    x2 = jnp.maximum(jnp.dot(x1, w0, preferred_element_type=jnp.float32) + b0, 0.0)
    x3 = jnp.maximum(jnp.dot(x2, w0, preferred_element_type=jnp.float32) + b0, 0.0)
    o_ref[...] = jnp.dot(x3, w4_ref[...], preferred_element_type=jnp.float32)


def _mlp_stage(partials, W0, b0, W4):
    return pl.pallas_call(
        _mlp_body,
        grid=(N_NODES // BN,),
        in_specs=[
            pl.BlockSpec((1, BN, EMB), lambda i: (0, i, 0)),
            pl.BlockSpec((1, BN, EMB), lambda i: (1, i, 0)),
            pl.BlockSpec((EMB, EMB), lambda i: (0, 0)),
            pl.BlockSpec((1, EMB), lambda i: (0, 0)),
            pl.BlockSpec((EMB, EMB), lambda i: (0, 0)),
        ],
        out_specs=pl.BlockSpec((BN, EMB), lambda i: (i, 0)),
        out_shape=jax.ShapeDtypeStruct((N_NODES, EMB), jnp.float32),
    )(partials, partials, W0, b0, W4)


def kernel(edge_attr, edge_index, msg_emb, num_nodes, W_edge, b_edge, W0, b0, W4):
    x = _edge_stage(edge_attr.T, msg_emb, W_edge, b_edge.reshape(1, EMB))
    partials = _sc_scatter()(x, edge_index)
    return _mlp_stage(partials, W0, b0.reshape(1, EMB), W4)


# speedup vs baseline: 1.0500x; 1.0251x over previous
import functools

import jax
import jax.numpy as jnp
from jax import lax
from jax.experimental import pallas as pl
from jax.experimental.pallas import tpu as pltpu
from jax.experimental.pallas import tpu_sc as plsc

N_NODES = 10000
N_EDGES = 320000
D_EDGE = 16
EMB = 128

NC = 2
NS = 16
NW = NC * NS
CH = 128
NCHUNK = N_EDGES // CH
KFULL = NCHUNK // NW
NEXTRA = NCHUNK - KFULL * NW
NB = 2
NFULL = KFULL // NB
NP = 10240
RPT = NP // NS
NLANE = 16

BE = 12800
BN = 2000


def _edge_body(attr_ref, msg_ref, w_ref, b_ref, o_ref):
    emb = lax.dot_general(attr_ref[...], w_ref[...],
                          dimension_numbers=(((0,), (0,)), ((), ())),
                          preferred_element_type=jnp.float32)
    o_ref[...] = (emb + b_ref[...]) * msg_ref[...]


def _edge_stage(edge_attr_t, msg_emb, W_edge, b_edge):
    return pl.pallas_call(
        _edge_body,
        grid=(N_EDGES // BE,),
        in_specs=[
            pl.BlockSpec((D_EDGE, BE), lambda i: (0, i)),
            pl.BlockSpec((BE, EMB), lambda i: (i, 0)),
            pl.BlockSpec((D_EDGE, EMB), lambda i: (0, 0)),
            pl.BlockSpec((1, EMB), lambda i: (0, 0)),
        ],
        out_specs=pl.BlockSpec((BE, EMB), lambda i: (i, 0)),
        out_shape=jax.ShapeDtypeStruct((N_EDGES, EMB), jnp.float32),
    )(edge_attr_t, msg_emb, W_edge, b_edge)


def _sc_body(x_hbm, eidx_hbm, out_hbm, xbs, idbs, acc, xsems, isems):
    c = lax.axis_index("c")
    s = lax.axis_index("s")
    wid = s * NC + c

    def chunk_of(k):
        return lax.select(k < KFULL, k * NW + wid, KFULL * NW + wid)

    def start_fetch(k, t):
        ch = chunk_of(k)
        pltpu.async_copy(x_hbm.at[pl.ds(ch * CH, CH)], xbs[t], xsems[t])
        pltpu.async_copy(eidx_hbm.at[:, pl.ds(ch * CH, CH)], idbs[t], isems[t])

    def finish_chunk(k, t):
        ch = chunk_of(k)
        pltpu.make_async_copy(
            x_hbm.at[pl.ds(ch * CH, CH)], xbs[t], xsems[t]).wait()
        pltpu.make_async_copy(
            eidx_hbm.at[:, pl.ds(ch * CH, CH)], idbs[t], isems[t]).wait()
        pltpu.sync_copy(xbs[t], acc.at[idbs[t].at[1]], add=True)

    nk = KFULL + lax.select(wid < NEXTRA, 1, 0)

    zv = jnp.zeros((NLANE,), jnp.float32)

    def zrow(r, carry):
        for j in range(EMB // NLANE):
            xbs[0][r, pl.ds(j * NLANE, NLANE)] = zv
        return carry

    lax.fori_loop(0, CH, zrow, 0)
    for rep in range(RPT // CH):
        pltpu.sync_copy(xbs[0], acc.at[pl.ds(s * RPT + rep * CH, CH)])

    for t in range(NB):
        start_fetch(t, t)
    plsc.subcore_barrier()

    def body(i, carry):
        for t in range(NB):
            k = i * NB + t
            finish_chunk(k, t)

            @pl.when(k + NB < nk)
            def _():
                start_fetch(k + NB, t)

        return carry

    lax.fori_loop(0, NFULL, body, 0)

    @pl.when(nk > KFULL)
    def _():
        finish_chunk(KFULL, KFULL % NB)

    plsc.subcore_barrier()
    pltpu.sync_copy(acc.at[pl.ds(s * RPT, RPT)],
                    out_hbm.at[c, pl.ds(s * RPT, RPT)])


@functools.cache
def _sc_scatter():
    return pl.kernel(
        _sc_body,
        out_type=jax.ShapeDtypeStruct((NC, NP, EMB), jnp.float32),
        mesh=plsc.VectorSubcoreMesh(
            core_axis_name="c", subcore_axis_name="s", num_cores=NC, num_subcores=NS
        ),
        scratch_types=[
            tuple(pltpu.VMEM((CH, EMB), jnp.float32) for _ in range(NB)),
            tuple(pltpu.VMEM((2, CH), jnp.int32) for _ in range(NB)),
            pltpu.VMEM_SHARED((NP, EMB), jnp.float32),
            tuple(pltpu.SemaphoreType.DMA for _ in range(NB)),
            tuple(pltpu.SemaphoreType.DMA for _ in range(NB)),
        ],
    )


def _mlp_body(p0_ref, p1_ref, w0_ref, b0_ref, w4_ref, o_ref):
    h = p0_ref[0] + p1_ref[0]
    w0 = w0_ref[...]
    b0 = b0_ref[...]
    x1 = jnp.maximum(jnp.dot(h, w0, preferred_element_type=jnp.float32) + b0, 0.0)
    x2 = jnp.maximum(jnp.dot(x1, w0, preferred_element_type=jnp.float32) + b0, 0.0)
    x3 = jnp.maximum(jnp.dot(x2, w0, preferred_element_type=jnp.float32) + b0, 0.0)
    o_ref[...] = jnp.dot(x3, w4_ref[...], preferred_element_type=jnp.float32)


def _mlp_stage(partials, W0, b0, W4):
    return pl.pallas_call(
        _mlp_body,
        grid=(N_NODES // BN,),
        in_specs=[
            pl.BlockSpec((1, BN, EMB), lambda i: (0, i, 0)),
            pl.BlockSpec((1, BN, EMB), lambda i: (1, i, 0)),
            pl.BlockSpec((EMB, EMB), lambda i: (0, 0)),
            pl.BlockSpec((1, EMB), lambda i: (0, 0)),
            pl.BlockSpec((EMB, EMB), lambda i: (0, 0)),
        ],
        out_specs=pl.BlockSpec((BN, EMB), lambda i: (i, 0)),
        out_shape=jax.ShapeDtypeStruct((N_NODES, EMB), jnp.float32),
    )(partials, partials, W0, b0, W4)


def kernel(edge_attr, edge_index, msg_emb, num_nodes, W_edge, b_edge, W0, b0, W4):
    x = _edge_stage(edge_attr.T, msg_emb, W_edge, b_edge.reshape(1, EMB))
    partials = _sc_scatter()(x, edge_index)
    return _mlp_stage(partials, W0, b0.reshape(1, EMB), W4)


# speedup vs baseline: 1.0536x; 1.0034x over previous
import functools

import jax
import jax.numpy as jnp
from jax import lax
from jax.experimental import pallas as pl
from jax.experimental.pallas import tpu as pltpu
from jax.experimental.pallas import tpu_sc as plsc

N_NODES = 10000
N_EDGES = 320000
D_EDGE = 16
EMB = 128

NC = 2
NS = 16
NW = NC * NS
CH = 128
NCHUNK = N_EDGES // CH
KFULL = NCHUNK // NW
NEXTRA = NCHUNK - KFULL * NW
NB = 2
NFULL = KFULL // NB
NP = 10240
RPT = NP // NS
NLANE = 16

BE = 16000
BN = 2000


def _edge_body(attr_ref, msg_ref, w_ref, b_ref, o_ref):
    emb = lax.dot_general(attr_ref[...], w_ref[...],
                          dimension_numbers=(((0,), (0,)), ((), ())),
                          preferred_element_type=jnp.float32)
    o_ref[...] = (emb + b_ref[...]) * msg_ref[...]


def _edge_stage(edge_attr_t, msg_emb, W_edge, b_edge):
    return pl.pallas_call(
        _edge_body,
        grid=(N_EDGES // BE,),
        in_specs=[
            pl.BlockSpec((D_EDGE, BE), lambda i: (0, i)),
            pl.BlockSpec((BE, EMB), lambda i: (i, 0)),
            pl.BlockSpec((D_EDGE, EMB), lambda i: (0, 0)),
            pl.BlockSpec((1, EMB), lambda i: (0, 0)),
        ],
        out_specs=pl.BlockSpec((BE, EMB), lambda i: (i, 0)),
        out_shape=jax.ShapeDtypeStruct((N_EDGES, EMB), jnp.float32),
    )(edge_attr_t, msg_emb, W_edge, b_edge)


def _sc_body(x_hbm, eidx_hbm, out_hbm, xbs, idbs, acc, xsems, isems):
    c = lax.axis_index("c")
    s = lax.axis_index("s")
    wid = s * NC + c

    def chunk_of(k):
        return lax.select(k < KFULL, k * NW + wid, KFULL * NW + wid)

    def start_fetch(k, t):
        ch = chunk_of(k)
        pltpu.async_copy(x_hbm.at[pl.ds(ch * CH, CH)], xbs[t], xsems[t])
        pltpu.async_copy(eidx_hbm.at[:, pl.ds(ch * CH, CH)], idbs[t], isems[t])

    def finish_chunk(k, t):
        ch = chunk_of(k)
        pltpu.make_async_copy(
            x_hbm.at[pl.ds(ch * CH, CH)], xbs[t], xsems[t]).wait()
        pltpu.make_async_copy(
            eidx_hbm.at[:, pl.ds(ch * CH, CH)], idbs[t], isems[t]).wait()
        pltpu.sync_copy(xbs[t], acc.at[idbs[t].at[1]], add=True)

    nk = KFULL + lax.select(wid < NEXTRA, 1, 0)

    zv = jnp.zeros((NLANE,), jnp.float32)

    def zrow(r, carry):
        for j in range(EMB // NLANE):
            xbs[0][r, pl.ds(j * NLANE, NLANE)] = zv
        return carry

    lax.fori_loop(0, CH, zrow, 0)
    for rep in range(RPT // CH):
        pltpu.sync_copy(xbs[0], acc.at[pl.ds(s * RPT + rep * CH, CH)])

    for t in range(NB):
        start_fetch(t, t)
    plsc.subcore_barrier()

    def body(i, carry):
        for t in range(NB):
            k = i * NB + t
            finish_chunk(k, t)

            @pl.when(k + NB < nk)
            def _():
                start_fetch(k + NB, t)

        return carry

    lax.fori_loop(0, NFULL, body, 0)

    @pl.when(nk > KFULL)
    def _():
        finish_chunk(KFULL, KFULL % NB)

    plsc.subcore_barrier()
    pltpu.sync_copy(acc.at[pl.ds(s * RPT, RPT)],
                    out_hbm.at[c, pl.ds(s * RPT, RPT)])


@functools.cache
def _sc_scatter():
    return pl.kernel(
        _sc_body,
        out_type=jax.ShapeDtypeStruct((NC, NP, EMB), jnp.float32),
        mesh=plsc.VectorSubcoreMesh(
            core_axis_name="c", subcore_axis_name="s", num_cores=NC, num_subcores=NS
        ),
        scratch_types=[
            tuple(pltpu.VMEM((CH, EMB), jnp.float32) for _ in range(NB)),
            tuple(pltpu.VMEM((2, CH), jnp.int32) for _ in range(NB)),
            pltpu.VMEM_SHARED((NP, EMB), jnp.float32),
            tuple(pltpu.SemaphoreType.DMA for _ in range(NB)),
            tuple(pltpu.SemaphoreType.DMA for _ in range(NB)),
        ],
    )


def _mlp_body(p0_ref, p1_ref, w0_ref, b0_ref, w4_ref, o_ref):
    h = p0_ref[0] + p1_ref[0]
    w0 = w0_ref[...]
    b0 = b0_ref[...]
    x1 = jnp.maximum(jnp.dot(h, w0, preferred_element_type=jnp.float32) + b0, 0.0)
    x2 = jnp.maximum(jnp.dot(x1, w0, preferred_element_type=jnp.float32) + b0, 0.0)
    x3 = jnp.maximum(jnp.dot(x2, w0, preferred_element_type=jnp.float32) + b0, 0.0)
    o_ref[...] = jnp.dot(x3, w4_ref[...], preferred_element_type=jnp.float32)


def _mlp_stage(partials, W0, b0, W4):
    return pl.pallas_call(
        _mlp_body,
        grid=(N_NODES // BN,),
        in_specs=[
            pl.BlockSpec((1, BN, EMB), lambda i: (0, i, 0)),
            pl.BlockSpec((1, BN, EMB), lambda i: (1, i, 0)),
            pl.BlockSpec((EMB, EMB), lambda i: (0, 0)),
            pl.BlockSpec((1, EMB), lambda i: (0, 0)),
            pl.BlockSpec((EMB, EMB), lambda i: (0, 0)),
        ],
        out_specs=pl.BlockSpec((BN, EMB), lambda i: (i, 0)),
        out_shape=jax.ShapeDtypeStruct((N_NODES, EMB), jnp.float32),
    )(partials, partials, W0, b0, W4)


def kernel(edge_attr, edge_index, msg_emb, num_nodes, W_edge, b_edge, W0, b0, W4):
    x = _edge_stage(edge_attr.T, msg_emb, W_edge, b_edge.reshape(1, EMB))
    partials = _sc_scatter()(x, edge_index)
    return _mlp_stage(partials, W0, b0.reshape(1, EMB), W4)
